# baseline (speedup 1.0000x reference)
import functools

import jax
import jax.numpy as jnp
from jax import lax
from jax.experimental import pallas as pl
from jax.experimental.pallas import tpu as pltpu
from jax.experimental.pallas import tpu_sc as plsc

_NC = 2
_NS = 16
_LANES = 16


def _make_sc_segsum(n, chunk, d, num_seg):
  nw = _NC * _NS
  num_chunks = n // chunk
  chunks_per_tile = (num_chunks + nw - 1) // nw
  mesh = plsc.VectorSubcoreMesh(core_axis_name="c", subcore_axis_name="s",
                                num_cores=_NC, num_subcores=_NS)

  @functools.partial(
      pl.kernel,
      out_type=[
          jax.ShapeDtypeStruct((_NC, num_seg, d), jnp.float32),
          jax.ShapeDtypeStruct((_NC, num_seg, d), jnp.float32),
      ],
      mesh=mesh,
      scratch_types=[
          pltpu.VMEM((2, chunk, d), jnp.float32),
          pltpu.VMEM((chunks_per_tile, chunk), jnp.int32),
          pltpu.VMEM((chunk, d), jnp.float32),
          pltpu.VMEM((num_seg, d), jnp.float32),
          pltpu.VMEM_SHARED((num_seg, d), jnp.float32),
          pltpu.VMEM_SHARED((num_seg, d), jnp.float32),
          pltpu.SemaphoreType.DMA,
          pltpu.SemaphoreType.DMA,
          pltpu.SemaphoreType.DMA,
          pltpu.SemaphoreType.DMA,
          pltpu.SemaphoreType.DMA,
          pltpu.SemaphoreType.DMA,
      ],
  )
  def sc_segsum(z_hbm, b_hbm, sums_out, cnts_out, rows_v, idx_v, ones_v,
                zeros_v, acc_sh, cnt_sh, sg0, sg1, ss0, ss1, so0, so1):
    cid = lax.axis_index("c")
    sid = lax.axis_index("s")
    wid = cid * _NS + sid
    sg = (sg0, sg1)
    ss = (ss0, ss1)
    so = (so0, so1)

    def valid(j):
      return (j * nw + wid) < num_chunks

    def zsrc(j):
      return z_hbm.at[pl.ds((j * nw + wid) * chunk, chunk), :]

    def bsrc(j):
      return b_hbm.at[pl.ds((j * nw + wid) * chunk, chunk)]

    ones = jnp.ones((_LANES,), jnp.float32)

    def fill_ones(i, _):
      for k in range(d // _LANES):
        ones_v[i, pl.ds(k * _LANES, _LANES)] = ones
      return 0

    lax.fori_loop(0, chunk, fill_ones, 0)

    @pl.when(valid(0))
    def _prefetch0():
      pltpu.async_copy(zsrc(0), rows_v.at[0], sg[0])
      pltpu.async_copy(bsrc(0), idx_v.at[0], sg[0])

    @pl.when(sid == 0)
    def _init():
      zeros = jnp.zeros((_LANES,), jnp.float32)

      def fill_zeros(i, _):
        for k in range(d // _LANES):
          zeros_v[i, pl.ds(k * _LANES, _LANES)] = zeros
        return 0

      lax.fori_loop(0, num_seg, fill_zeros, 0)
      pltpu.sync_copy(zeros_v, acc_sh)
      pltpu.sync_copy(zeros_v, cnt_sh)

    plsc.subcore_barrier()

    for j in range(chunks_per_tile):
      buf = j & 1

      @pl.when(valid(j))
      def _scatter_j(j=j, buf=buf):
        pltpu.make_async_copy(zsrc(j), rows_v.at[buf], sg[buf]).wait()
        pltpu.make_async_copy(bsrc(j), idx_v.at[j], sg[buf]).wait()
        pltpu.async_copy(rows_v.at[buf], acc_sh.at[idx_v.at[j]], ss[buf],
                         add=True)
        pltpu.async_copy(ones_v, cnt_sh.at[idx_v.at[j]], so[buf], add=True)

      if j + 1 < chunks_per_tile:

        @pl.when(valid(j + 1))
        def _prefetch_next(j=j):
          nbuf = (j + 1) & 1
          if j - 1 >= 0:
            pltpu.make_async_copy(rows_v.at[nbuf],
                                  acc_sh.at[idx_v.at[j - 1]],
                                  ss[nbuf]).wait()
            pltpu.make_async_copy(ones_v, cnt_sh.at[idx_v.at[j - 1]],
                                  so[nbuf]).wait()
          pltpu.async_copy(zsrc(j + 1), rows_v.at[nbuf], sg[nbuf])
          pltpu.async_copy(bsrc(j + 1), idx_v.at[j + 1], sg[nbuf])

    for j in range(max(chunks_per_tile - 2, 0), chunks_per_tile):
      buf = j & 1

      @pl.when(valid(j))
      def _drain_j(j=j, buf=buf):
        pltpu.make_async_copy(rows_v.at[buf], acc_sh.at[idx_v.at[j]],
                              ss[buf]).wait()
        pltpu.make_async_copy(ones_v, cnt_sh.at[idx_v.at[j]],
                              so[buf]).wait()

    plsc.subcore_barrier()

    @pl.when(sid == 0)
    def _writeout():
      pltpu.sync_copy(acc_sh, sums_out.at[cid])
      pltpu.sync_copy(cnt_sh, cnts_out.at[cid])

  return sc_segsum


def _gru_body(z_ref, x_ref, h_ref, wp_ref, bp_ref,
              wxz_ref, bxz_ref, whz_ref, bhz_ref,
              wxr_ref, bxr_ref, whr_ref, bhr_ref,
              wxh_ref, bxh_ref, whh_ref, bhh_ref,
              u_ref, wg_ref, bg_ref, out_ref, glob_ref,
              wx_s, whzr_s):
  f32 = jnp.float32
  db = whh_ref.shape[0]

  def dot(a, w):
    return jnp.dot(a, w, preferred_element_type=f32)

  @pl.when(pl.program_id(0) == 0)
  def _stage_weights():
    wx_s[:, 0:db] = wxz_ref[...]
    wx_s[:, db:2 * db] = wxr_ref[...]
    wx_s[:, 2 * db:3 * db] = wxh_ref[...]
    whzr_s[:, 0:db] = whz_ref[...]
    whzr_s[:, db:2 * db] = whr_ref[...]
    glob_ref[...] = jnp.maximum(
        dot(u_ref[...], wg_ref[...]) + bg_ref[...][None, :], 0.0)

  z = z_ref[...]
  h = h_ref[...]
  xp = jnp.maximum(dot(x_ref[...], wp_ref[...]) + bp_ref[...][None, :], 0.0)
  gin = jnp.concatenate([z, xp], axis=1)
  a = dot(gin, wx_s[...])
  ah = dot(h, whzr_s[...])
  zg = jax.nn.sigmoid(a[:, :db] + ah[:, :db]
                      + (bxz_ref[...] + bhz_ref[...])[None, :])
  rg = jax.nn.sigmoid(a[:, db:2 * db] + ah[:, db:2 * db]
                      + (bxr_ref[...] + bhr_ref[...])[None, :])
  ht = jnp.tanh(a[:, 2 * db:] + dot(rg * h, whh_ref[...])
                + (bxh_ref[...] + bhh_ref[...])[None, :])
  out_ref[...] = zg * h + (1.0 - zg) * ht


def _fused_body(s_ref, c_ref, glob_ref, out_ref):
  s = s_ref[0] + s_ref[1]
  cnt = c_ref[0, :, 0:1] + c_ref[1, :, 0:1]
  ge = s / jnp.maximum(cnt, 1.0)
  out_ref[...] = jnp.concatenate([ge, glob_ref[...]], axis=1)


def kernel(z, u, x, edge_index, batch, batch_size, prev_h, Wp, bp, Wg, bg,
           W_xz, b_xz, W_hz, b_hz, W_xr, b_xr, W_hr, b_hr, W_xh, b_xh,
           W_hh, b_hh):
  n, db = z.shape
  df = x.shape[1]
  dp = Wp.shape[1]
  b = u.shape[0]
  gin_d = db + dp

  chunk = 1
  for c in range(min(128, n), 0, -1):
    if n % c == 0 and c % 8 == 0:
      chunk = c
      break
  batch = batch.astype(jnp.int32)
  sums, cnts = _make_sc_segsum(n, chunk, db, b)(z, batch)

  blk = 5000
  grid = (n // blk,)
  row_spec = lambda width: pl.BlockSpec((blk, width), lambda i: (i, 0))
  full = lambda s: pl.BlockSpec(s, lambda i: (0,) * len(s))
  go = Wg.shape[1]
  h_new, glob = pl.pallas_call(
      _gru_body,
      grid=grid,
      in_specs=[
          row_spec(db), row_spec(df), row_spec(db),
          full((df, dp)), full((dp,)),
          full((gin_d, db)), full((db,)), full((db, db)), full((db,)),
          full((gin_d, db)), full((db,)), full((db, db)), full((db,)),
          full((gin_d, db)), full((db,)), full((db, db)), full((db,)),
          full((b, u.shape[1])), full((Wg.shape[0], go)), full((go,)),
      ],
      out_specs=[row_spec(db), full((b, go))],
      out_shape=[
          jax.ShapeDtypeStruct((n, db), jnp.float32),
          jax.ShapeDtypeStruct((b, go), jnp.float32),
      ],
      scratch_shapes=[
          pltpu.VMEM((gin_d, 3 * db), jnp.float32),
          pltpu.VMEM((db, 2 * db), jnp.float32),
      ],
  )(z, x, prev_h, Wp, bp,
    W_xz, b_xz, W_hz, b_hz,
    W_xr, b_xr, W_hr, b_hr,
    W_xh, b_xh, W_hh, b_hh,
    u, Wg, bg)

  fused = pl.pallas_call(
      _fused_body,
      out_shape=jax.ShapeDtypeStruct((b, db + go), jnp.float32),
  )(sums, cnts, glob)

  return (fused, h_new)


# speedup vs baseline: 1.0152x; 1.0152x over previous
import functools

import jax
import jax.numpy as jnp
from jax import lax
from jax.experimental import pallas as pl
from jax.experimental.pallas import tpu as pltpu
from jax.experimental.pallas import tpu_sc as plsc

_NC = 2
_NS = 16
_LANES = 16


def _make_sc_segsum(n, chunk, d, num_seg):
  nw = _NC * _NS
  num_chunks = n // chunk
  chunks_per_tile = (num_chunks + nw - 1) // nw
  mesh = plsc.VectorSubcoreMesh(core_axis_name="c", subcore_axis_name="s",
                                num_cores=_NC, num_subcores=_NS)

  @functools.partial(
      pl.kernel,
      out_type=[
          jax.ShapeDtypeStruct((_NC, num_seg, d), jnp.float32),
          jax.ShapeDtypeStruct((_NC, num_seg, d), jnp.float32),
      ],
      mesh=mesh,
      scratch_types=[
          pltpu.VMEM((2, chunk, d), jnp.float32),
          pltpu.VMEM((chunks_per_tile, chunk), jnp.int32),
          pltpu.VMEM((chunk, d), jnp.float32),
          pltpu.VMEM((num_seg, d), jnp.float32),
          pltpu.VMEM_SHARED((num_seg, d), jnp.float32),
          pltpu.VMEM_SHARED((num_seg, d), jnp.float32),
          pltpu.SemaphoreType.DMA,
          pltpu.SemaphoreType.DMA,
          pltpu.SemaphoreType.DMA,
          pltpu.SemaphoreType.DMA,
          pltpu.SemaphoreType.DMA,
          pltpu.SemaphoreType.DMA,
      ],
  )
  def sc_segsum(z_hbm, b_hbm, sums_out, cnts_out, rows_v, idx_v, ones_v,
                zeros_v, acc_sh, cnt_sh, sg0, sg1, ss0, ss1, so0, so1):
    cid = lax.axis_index("c")
    sid = lax.axis_index("s")
    wid = cid * _NS + sid
    sg = (sg0, sg1)
    ss = (ss0, ss1)
    so = (so0, so1)

    def valid(j):
      return (j * nw + wid) < num_chunks

    def zsrc(j):
      return z_hbm.at[pl.ds((j * nw + wid) * chunk, chunk), :]

    def bsrc(j):
      return b_hbm.at[pl.ds((j * nw + wid) * chunk, chunk)]

    ones = jnp.ones((_LANES,), jnp.float32)

    def fill_ones(i, _):
      for k in range(d // _LANES):
        ones_v[i, pl.ds(k * _LANES, _LANES)] = ones
      return 0

    lax.fori_loop(0, chunk, fill_ones, 0)

    @pl.when(valid(0))
    def _prefetch0():
      pltpu.async_copy(zsrc(0), rows_v.at[0], sg[0])
      pltpu.async_copy(bsrc(0), idx_v.at[0], sg[0])

    @pl.when(sid == 0)
    def _init():
      zeros = jnp.zeros((_LANES,), jnp.float32)

      def fill_zeros(i, _):
        for k in range(d // _LANES):
          zeros_v[i, pl.ds(k * _LANES, _LANES)] = zeros
        return 0

      lax.fori_loop(0, num_seg, fill_zeros, 0)
      pltpu.sync_copy(zeros_v, acc_sh)
      pltpu.sync_copy(zeros_v, cnt_sh)

    plsc.subcore_barrier()

    for j in range(chunks_per_tile):
      buf = j & 1

      @pl.when(valid(j))
      def _scatter_j(j=j, buf=buf):
        pltpu.make_async_copy(zsrc(j), rows_v.at[buf], sg[buf]).wait()
        pltpu.make_async_copy(bsrc(j), idx_v.at[j], sg[buf]).wait()
        pltpu.async_copy(rows_v.at[buf], acc_sh.at[idx_v.at[j]], ss[buf],
                         add=True)
        pltpu.async_copy(ones_v, cnt_sh.at[idx_v.at[j]], so[buf], add=True)

      if j + 1 < chunks_per_tile:

        @pl.when(valid(j + 1))
        def _prefetch_next(j=j):
          nbuf = (j + 1) & 1
          if j - 1 >= 0:
            pltpu.make_async_copy(rows_v.at[nbuf],
                                  acc_sh.at[idx_v.at[j - 1]],
                                  ss[nbuf]).wait()
            pltpu.make_async_copy(ones_v, cnt_sh.at[idx_v.at[j - 1]],
                                  so[nbuf]).wait()
          pltpu.async_copy(zsrc(j + 1), rows_v.at[nbuf], sg[nbuf])
          pltpu.async_copy(bsrc(j + 1), idx_v.at[j + 1], sg[nbuf])

    for j in range(max(chunks_per_tile - 2, 0), chunks_per_tile):
      buf = j & 1

      @pl.when(valid(j))
      def _drain_j(j=j, buf=buf):
        pltpu.make_async_copy(rows_v.at[buf], acc_sh.at[idx_v.at[j]],
                              ss[buf]).wait()
        pltpu.make_async_copy(ones_v, cnt_sh.at[idx_v.at[j]],
                              so[buf]).wait()

    plsc.subcore_barrier()

    @pl.when(sid == 0)
    def _writeout():
      pltpu.sync_copy(acc_sh, sums_out.at[cid])
      pltpu.sync_copy(cnt_sh, cnts_out.at[cid])

  return sc_segsum


def _gru_body(z_ref, x_ref, h_ref, wp_ref, bp_ref,
              wxz_ref, bxz_ref, whz_ref, bhz_ref,
              wxr_ref, bxr_ref, whr_ref, bhr_ref,
              wxh_ref, bxh_ref, whh_ref, bhh_ref, out_ref,
              wx_s, whzr_s):
  f32 = jnp.float32
  db = whh_ref.shape[0]

  def dot(a, w):
    return jnp.dot(a, w, preferred_element_type=f32)

  @pl.when(pl.program_id(0) == 0)
  def _stage_weights():
    wx_s[:, 0:db] = wxz_ref[...]
    wx_s[:, db:2 * db] = wxr_ref[...]
    wx_s[:, 2 * db:3 * db] = wxh_ref[...]
    whzr_s[:, 0:db] = whz_ref[...]
    whzr_s[:, db:2 * db] = whr_ref[...]

  z = z_ref[...]
  h = h_ref[...]
  xp = jnp.maximum(dot(x_ref[...], wp_ref[...]) + bp_ref[...][None, :], 0.0)
  gin = jnp.concatenate([z, xp], axis=1)
  a = dot(gin, wx_s[...])
  ah = dot(h, whzr_s[...])
  zg = jax.nn.sigmoid(a[:, :db] + ah[:, :db]
                      + (bxz_ref[...] + bhz_ref[...])[None, :])
  rg = jax.nn.sigmoid(a[:, db:2 * db] + ah[:, db:2 * db]
                      + (bxr_ref[...] + bhr_ref[...])[None, :])
  ht = jnp.tanh(a[:, 2 * db:] + dot(rg * h, whh_ref[...])
                + (bxh_ref[...] + bhh_ref[...])[None, :])
  out_ref[...] = zg * h + (1.0 - zg) * ht


def _fused_body(s_ref, c_ref, u_ref, wg_ref, bg_ref, out_ref):
  s = s_ref[0] + s_ref[1]
  cnt = c_ref[0, :, 0:1] + c_ref[1, :, 0:1]
  ge = s / jnp.maximum(cnt, 1.0)
  glob = jnp.maximum(
      jnp.dot(u_ref[...], wg_ref[...], preferred_element_type=jnp.float32)
      + bg_ref[...][None, :], 0.0)
  out_ref[...] = jnp.concatenate([ge, glob], axis=1)


def kernel(z, u, x, edge_index, batch, batch_size, prev_h, Wp, bp, Wg, bg,
           W_xz, b_xz, W_hz, b_hz, W_xr, b_xr, W_hr, b_hr, W_xh, b_xh,
           W_hh, b_hh):
  n, db = z.shape
  df = x.shape[1]
  dp = Wp.shape[1]
  b = u.shape[0]
  gin_d = db + dp

  chunk = 1
  for c in range(min(128, n), 0, -1):
    if n % c == 0 and c % 8 == 0:
      chunk = c
      break
  batch = batch.astype(jnp.int32)
  sums, cnts = _make_sc_segsum(n, chunk, db, b)(z, batch)

  blk = 5000
  grid = (n // blk,)
  row_spec = lambda width: pl.BlockSpec((blk, width), lambda i: (i, 0))
  full = lambda s: pl.BlockSpec(s, lambda i: (0,) * len(s))
  h_new = pl.pallas_call(
      _gru_body,
      grid=grid,
      in_specs=[
          row_spec(db), row_spec(df), row_spec(db),
          full((df, dp)), full((dp,)),
          full((gin_d, db)), full((db,)), full((db, db)), full((db,)),
          full((gin_d, db)), full((db,)), full((db, db)), full((db,)),
          full((gin_d, db)), full((db,)), full((db, db)), full((db,)),
      ],
      out_specs=row_spec(db),
      out_shape=jax.ShapeDtypeStruct((n, db), jnp.float32),
      scratch_shapes=[
          pltpu.VMEM((gin_d, 3 * db), jnp.float32),
          pltpu.VMEM((db, 2 * db), jnp.float32),
      ],
  )(z, x, prev_h, Wp, bp,
    W_xz, b_xz, W_hz, b_hz,
    W_xr, b_xr, W_hr, b_hr,
    W_xh, b_xh, W_hh, b_hh)

  go = Wg.shape[1]
  fused = pl.pallas_call(
      _fused_body,
      out_shape=jax.ShapeDtypeStruct((b, db + go), jnp.float32),
  )(sums, cnts, u, Wg, bg)

  return (fused, h_new)
